# baseline (speedup 1.0000x reference)
import functools

import jax
import jax.numpy as jnp
from jax import lax
from jax.experimental import pallas as pl
from jax.experimental.pallas import tpu as pltpu
from jax.experimental.pallas import tpu_sc as plsc

NC = 2
NS = 16
L = 16
NW = NC * NS

B, C, V = 1024, 128, 200
VP = 256
ROWS = B * C
ROWS_PER_W = ROWS // NW
R = 128
NCHUNK = ROWS_PER_W // R
NG = VP // L


def _body(x_hbm, idx_hbm, out_hbm, idx_v, in_v, out_v):
  wid = lax.axis_index("s") * NC + lax.axis_index("c")
  row0 = wid * ROWS_PER_W

  pltpu.sync_copy(idx_hbm, idx_v)

  idx_g = [idx_v[pl.ds(g * L, L)] for g in range(NG)]
  mask_g = [ig < V for ig in idx_g]
  idxc_g = [jnp.minimum(ig, V - 1) for ig in idx_g]

  def chunk_body(c, _):
    r0 = row0 + c * R
    pltpu.sync_copy(x_hbm.at[pl.ds(r0 * V, R * V)], in_v)

    def row_body(r, _):
      for g in range(NG):
        vals = plsc.load_gather(in_v, [idxc_g[g] + r * V])
        vals = jnp.where(mask_g[g], vals, 0.0)
        out_v[pl.ds(r * VP + g * L, L)] = vals
      return 0

    lax.fori_loop(0, R, row_body, 0)
    pltpu.sync_copy(out_v, out_hbm.at[pl.ds(r0 * VP, R * VP)])
    return 0

  lax.fori_loop(0, NCHUNK, chunk_body, 0)


@jax.jit
def kernel(input, indices):
  x_flat = input.reshape(ROWS * V)
  mesh = plsc.VectorSubcoreMesh(
      core_axis_name="c", subcore_axis_name="s", num_cores=NC, num_subcores=NS)
  out = pl.kernel(
      _body,
      out_type=jax.ShapeDtypeStruct((ROWS * VP,), jnp.float32),
      mesh=mesh,
      compiler_params=pltpu.CompilerParams(needs_layout_passes=False),
      scratch_types=[
          pltpu.VMEM((VP,), jnp.int32),
          pltpu.VMEM((R * V,), jnp.float32),
          pltpu.VMEM((R * VP,), jnp.float32),
      ],
  )(x_flat, indices)
  return out.reshape(B, C, VP)
